# baseline (speedup 1.0000x reference)
import functools

import jax
import jax.numpy as jnp
from jax import lax
from jax.experimental import pallas as pl
from jax.experimental.pallas import tpu as pltpu
from jax.experimental.pallas import tpu_sc as plsc

B = 16384
NI = 1000000
D = 64
DP = 128
K = 20
LANES = 16

NC = 2
NS = 16
NW = NC * NS

BPW = B // NW
CB = 16
NCHUNK = BPW // CB
NIDX = CB * K
NSLICES = [(0, 128), (128, 128), (256, 64)]
NVR = NIDX // LANES

BT = 8192
SBLK = 62
S = SBLK * BT


def _tc_relayout_body(a_ref, b_ref, out_ref):
    ey = jnp.eye(D, dtype=jnp.float32)
    dims = (((0,), (0,)), ((), ()))
    ta = lax.dot_general(a_ref[...], ey, dims,
                         preferred_element_type=jnp.float32)
    tb = lax.dot_general(b_ref[...], ey, dims,
                         preferred_element_type=jnp.float32)
    out_ref[...] = jnp.concatenate([ta, tb], axis=1)


@functools.cache
def _build_tc_relayout():
    return pl.pallas_call(
        _tc_relayout_body,
        grid=(SBLK,),
        in_specs=[
            pl.BlockSpec((D, BT), lambda g: (0, g)),
            pl.BlockSpec((D, BT), lambda g: (0, jnp.minimum(g + SBLK, NI // BT))),
        ],
        out_specs=pl.BlockSpec((BT, DP), lambda g: (g, 0)),
        out_shape=jax.ShapeDtypeStruct((S, DP), jnp.float32),
    )


def _log_sigmoid(x):
    t = jnp.exp(-jnp.abs(x))
    z = t / (t + 2.0)
    z2 = z * z
    p = 1.0 + z2 * (1.0 / 3.0 + z2 * (1.0 / 5.0 + z2 * (1.0 / 7.0 + z2 * (1.0 / 9.0))))
    return jnp.minimum(x, 0.0) - 2.0 * z * p


def _sc_body(center_ref, pos_ref, neg_ref, ein_ref, eout_ref, out_ref,
             craw0, craw1, praw0, praw1, nraw0, nraw1,
             cdma0, cdma1, pdma0, pdma1, ndma0, ndma1,
             c0, c1, p0, p1, n0, n1, lacc, sem0, sem1):
    wid = lax.axis_index("s") * NC + lax.axis_index("c")
    craw = (craw0, craw1)
    praw = (praw0, praw1)
    nraw = (nraw0, nraw1)
    cdma = (cdma0, cdma1)
    pdma = (pdma0, pdma1)
    ndma = (ndma0, ndma1)
    crows = (c0, c1)
    prows = (p0, p1)
    nrows = (n0, n1)
    sems = (sem0, sem1)

    iota16 = lax.iota(jnp.int32, LANES)
    perms = [iota16 ^ sh for sh in (1, 2, 4, 8)]

    def hsum(t):
        for p in perms:
            t = t + jnp.take(t, p)
        return t

    def packed_row(v):
        return jnp.where(v >= S, v - S, v)

    lacc[...] = jnp.zeros((LANES,), jnp.float32)

    def issue(g, s):
        cbase = wid * BPW + g * CB
        pltpu.sync_copy(center_ref.at[pl.ds(cbase, CB)],
                        craw[s].at[pl.ds(0, CB)])
        pltpu.sync_copy(pos_ref.at[pl.ds(cbase, CB)],
                        praw[s].at[pl.ds(0, CB)])
        pltpu.sync_copy(neg_ref.at[pl.ds(cbase * K, NIDX)],
                        nraw[s].at[pl.ds(0, NIDX)])
        cdma[s][...] = packed_row(craw[s][pl.ds(0, LANES)])
        pdma[s][...] = packed_row(praw[s][pl.ds(0, LANES)])
        for i in range(NVR):
            ndma[s][pl.ds(i * LANES, LANES)] = packed_row(
                nraw[s][pl.ds(i * LANES, LANES)])
        pltpu.async_copy(ein_ref.at[cdma[s]], crows[s], sems[s])
        pltpu.async_copy(eout_ref.at[pdma[s]], prows[s], sems[s])
        for off, num in NSLICES:
            pltpu.async_copy(
                eout_ref.at[ndma[s].at[pl.ds(off, num)]],
                nrows[s].at[pl.ds(off, num)], sems[s])

    def drain(s):
        pltpu.make_async_copy(ein_ref.at[pl.ds(0, CB)], crows[s], sems[s]).wait()
        pltpu.make_async_copy(eout_ref.at[pl.ds(0, CB)], prows[s], sems[s]).wait()
        pltpu.make_async_copy(eout_ref.at[pl.ds(0, NIDX)], nrows[s], sems[s]).wait()

    def compute(s):
        cb, pb, nb = crows[s], prows[s], nrows[s]
        craw_s, praw_s, nraw_s = craw[s], praw[s], nraw[s]

        def b_body(bl, slots):
            mask = iota16 == bl

            ce = craw_s[pl.ds(bl, LANES)][0]
            pe = praw_s[pl.ds(bl, LANES)][0]
            nw1 = nraw_s[pl.ds(bl * K, LANES)]
            nw2 = nraw_s[pl.ds(bl * K + LANES, LANES)]

            def half(e):
                return jnp.where(e >= S, D, 0)

            coff = half(ce)
            cq = [cb[bl, pl.ds(coff + m * LANES, LANES)]
                  for m in range(D // LANES)]

            def dot_row(ref, r, off):
                t = cq[0] * ref[r, pl.ds(off, LANES)]
                for m in range(1, D // LANES):
                    t = t + cq[m] * ref[r, pl.ds(off + m * LANES, LANES)]
                return hsum(t)

            news = [None] * (K + 1)
            news[0] = jnp.where(mask, dot_row(pb, bl, half(pe)), slots[0])
            nbase = bl * K
            for k in range(K):
                ne = nw1[k] if k < LANES else nw2[k - LANES]
                news[1 + k] = jnp.where(
                    mask, dot_row(nb, nbase + k, half(ne)), slots[1 + k])
            return tuple(news)

        zero = jnp.zeros((LANES,), jnp.float32)
        slots = lax.fori_loop(0, CB, b_body, (zero,) * (K + 1))
        tot = _log_sigmoid(slots[0])
        for k in range(K):
            tot = tot + _log_sigmoid(-slots[1 + k])
        lacc[...] = lacc[...] + tot

    issue(0, 0)
    issue(1, 1)

    def t_body(i, carry):
        for s in (0, 1):
            g = i * 2 + s
            drain(s)
            compute(s)
            ng = g + 2

            @pl.when(ng < NCHUNK)
            def _():
                issue(ng, s)
        return carry

    lax.fori_loop(0, NCHUNK // 2, t_body, 0)
    pltpu.sync_copy(lacc, out_ref.at[wid])


@functools.cache
def _build_sc_kernel():
    mesh = plsc.VectorSubcoreMesh(
        core_axis_name="c", subcore_axis_name="s",
        num_cores=NC, num_subcores=NS)
    return pl.kernel(
        _sc_body,
        out_type=jax.ShapeDtypeStruct((NW, LANES), jnp.float32),
        mesh=mesh,
        scratch_types=[
            pltpu.VMEM((2 * CB,), jnp.int32), pltpu.VMEM((2 * CB,), jnp.int32),
            pltpu.VMEM((2 * CB,), jnp.int32), pltpu.VMEM((2 * CB,), jnp.int32),
            pltpu.VMEM((NIDX + LANES,), jnp.int32),
            pltpu.VMEM((NIDX + LANES,), jnp.int32),
            pltpu.VMEM((CB,), jnp.int32), pltpu.VMEM((CB,), jnp.int32),
            pltpu.VMEM((CB,), jnp.int32), pltpu.VMEM((CB,), jnp.int32),
            pltpu.VMEM((NIDX,), jnp.int32), pltpu.VMEM((NIDX,), jnp.int32),
            pltpu.VMEM((CB, DP), jnp.float32), pltpu.VMEM((CB, DP), jnp.float32),
            pltpu.VMEM((CB, DP), jnp.float32), pltpu.VMEM((CB, DP), jnp.float32),
            pltpu.VMEM((NIDX, DP), jnp.float32),
            pltpu.VMEM((NIDX, DP), jnp.float32),
            pltpu.VMEM((LANES,), jnp.float32),
            pltpu.SemaphoreType.DMA, pltpu.SemaphoreType.DMA,
        ],
    )


def kernel(center_ids, pos_ids, neg_ids, emb_in_weight, emb_out_weight):
    relayout = _build_tc_relayout()
    t_in = relayout(emb_in_weight.T, emb_in_weight.T)
    t_out = relayout(emb_out_weight.T, emb_out_weight.T)
    neg_flat = neg_ids.astype(jnp.int32).reshape(B * K)
    partials = _build_sc_kernel()(center_ids.astype(jnp.int32),
                                  pos_ids.astype(jnp.int32),
                                  neg_flat, t_in, t_out)
    return -jnp.sum(partials) / B


# speedup vs baseline: 1.0540x; 1.0540x over previous
import functools

import jax
import jax.numpy as jnp
from jax import lax
from jax.experimental import pallas as pl
from jax.experimental.pallas import tpu as pltpu
from jax.experimental.pallas import tpu_sc as plsc

B = 16384
NI = 1000000
D = 64
DP = 128
K = 20
LANES = 16

NC = 2
NS = 16
NW = NC * NS

BPW = B // NW
CB = 16
NCHUNK = BPW // CB
NIDX = CB * K
NSLICES = [(0, 128), (128, 128), (256, 64)]
NVR = NIDX // LANES

BT = 16384
SBLK = 31
S = SBLK * BT


def _tc_relayout_body(a_ref, b_ref, out_ref):
    ta = jnp.transpose(a_ref[...])
    tb = jnp.transpose(b_ref[...])
    out_ref[...] = jnp.concatenate([ta, tb], axis=1)


@functools.cache
def _build_tc_relayout():
    return pl.pallas_call(
        _tc_relayout_body,
        grid=(SBLK,),
        in_specs=[
            pl.BlockSpec((D, BT), lambda g: (0, g)),
            pl.BlockSpec((D, BT), lambda g: (0, jnp.minimum(g + SBLK, NI // BT))),
        ],
        out_specs=pl.BlockSpec((BT, DP), lambda g: (g, 0)),
        out_shape=jax.ShapeDtypeStruct((S, DP), jnp.float32),
    )


def _log_sigmoid(x):
    t = jnp.exp(-jnp.abs(x))
    z = t / (t + 2.0)
    z2 = z * z
    p = 1.0 + z2 * (1.0 / 3.0 + z2 * (1.0 / 5.0 + z2 * (1.0 / 7.0 + z2 * (1.0 / 9.0))))
    return jnp.minimum(x, 0.0) - 2.0 * z * p


def _sc_body(center_ref, pos_ref, neg_ref, ein_ref, eout_ref, out_ref,
             craw0, craw1, praw0, praw1, nraw0, nraw1,
             cdma0, cdma1, pdma0, pdma1, ndma0, ndma1,
             c0, c1, p0, p1, n0, n1, lacc, sem0, sem1):
    wid = lax.axis_index("s") * NC + lax.axis_index("c")
    craw = (craw0, craw1)
    praw = (praw0, praw1)
    nraw = (nraw0, nraw1)
    cdma = (cdma0, cdma1)
    pdma = (pdma0, pdma1)
    ndma = (ndma0, ndma1)
    crows = (c0, c1)
    prows = (p0, p1)
    nrows = (n0, n1)
    sems = (sem0, sem1)

    iota16 = lax.iota(jnp.int32, LANES)
    perms = [iota16 ^ sh for sh in (1, 2, 4, 8)]

    def hsum(t):
        for p in perms:
            t = t + jnp.take(t, p)
        return t

    def packed_row(v):
        return jnp.where(v >= S, v - S, v)

    lacc[...] = jnp.zeros((LANES,), jnp.float32)

    def issue(g, s):
        cbase = wid * BPW + g * CB
        pltpu.sync_copy(center_ref.at[pl.ds(cbase, CB)],
                        craw[s].at[pl.ds(0, CB)])
        pltpu.sync_copy(pos_ref.at[pl.ds(cbase, CB)],
                        praw[s].at[pl.ds(0, CB)])
        pltpu.sync_copy(neg_ref.at[pl.ds(cbase * K, NIDX)],
                        nraw[s].at[pl.ds(0, NIDX)])
        cdma[s][...] = packed_row(craw[s][pl.ds(0, LANES)])
        pdma[s][...] = packed_row(praw[s][pl.ds(0, LANES)])
        for i in range(NVR):
            ndma[s][pl.ds(i * LANES, LANES)] = packed_row(
                nraw[s][pl.ds(i * LANES, LANES)])
        pltpu.async_copy(ein_ref.at[cdma[s]], crows[s], sems[s])
        pltpu.async_copy(eout_ref.at[pdma[s]], prows[s], sems[s])
        for off, num in NSLICES:
            pltpu.async_copy(
                eout_ref.at[ndma[s].at[pl.ds(off, num)]],
                nrows[s].at[pl.ds(off, num)], sems[s])

    def drain(s):
        pltpu.make_async_copy(ein_ref.at[pl.ds(0, CB)], crows[s], sems[s]).wait()
        pltpu.make_async_copy(eout_ref.at[pl.ds(0, CB)], prows[s], sems[s]).wait()
        pltpu.make_async_copy(eout_ref.at[pl.ds(0, NIDX)], nrows[s], sems[s]).wait()

    def compute(s):
        cb, pb, nb = crows[s], prows[s], nrows[s]
        craw_s, praw_s, nraw_s = craw[s], praw[s], nraw[s]

        def b_body(bl, slots):
            mask = iota16 == bl

            ce = craw_s[pl.ds(bl, LANES)][0]
            pe = praw_s[pl.ds(bl, LANES)][0]
            nw1 = nraw_s[pl.ds(bl * K, LANES)]
            nw2 = nraw_s[pl.ds(bl * K + LANES, LANES)]

            def half(e):
                return jnp.where(e >= S, D, 0)

            coff = half(ce)
            cq = [cb[bl, pl.ds(coff + m * LANES, LANES)]
                  for m in range(D // LANES)]

            def dot_row(ref, r, off):
                t = cq[0] * ref[r, pl.ds(off, LANES)]
                for m in range(1, D // LANES):
                    t = t + cq[m] * ref[r, pl.ds(off + m * LANES, LANES)]
                return hsum(t)

            news = [None] * (K + 1)
            news[0] = jnp.where(mask, dot_row(pb, bl, half(pe)), slots[0])
            nbase = bl * K
            for k in range(K):
                ne = nw1[k] if k < LANES else nw2[k - LANES]
                news[1 + k] = jnp.where(
                    mask, dot_row(nb, nbase + k, half(ne)), slots[1 + k])
            return tuple(news)

        zero = jnp.zeros((LANES,), jnp.float32)
        slots = lax.fori_loop(0, CB, b_body, (zero,) * (K + 1))
        tot = _log_sigmoid(slots[0])
        for k in range(K):
            tot = tot + _log_sigmoid(-slots[1 + k])
        lacc[...] = lacc[...] + tot

    issue(0, 0)
    issue(1, 1)

    def t_body(i, carry):
        for s in (0, 1):
            g = i * 2 + s
            drain(s)
            compute(s)
            ng = g + 2

            @pl.when(ng < NCHUNK)
            def _():
                issue(ng, s)
        return carry

    lax.fori_loop(0, NCHUNK // 2, t_body, 0)
    pltpu.sync_copy(lacc, out_ref.at[wid])


@functools.cache
def _build_sc_kernel():
    mesh = plsc.VectorSubcoreMesh(
        core_axis_name="c", subcore_axis_name="s",
        num_cores=NC, num_subcores=NS)
    return pl.kernel(
        _sc_body,
        out_type=jax.ShapeDtypeStruct((NW, LANES), jnp.float32),
        mesh=mesh,
        scratch_types=[
            pltpu.VMEM((2 * CB,), jnp.int32), pltpu.VMEM((2 * CB,), jnp.int32),
            pltpu.VMEM((2 * CB,), jnp.int32), pltpu.VMEM((2 * CB,), jnp.int32),
            pltpu.VMEM((NIDX + LANES,), jnp.int32),
            pltpu.VMEM((NIDX + LANES,), jnp.int32),
            pltpu.VMEM((CB,), jnp.int32), pltpu.VMEM((CB,), jnp.int32),
            pltpu.VMEM((CB,), jnp.int32), pltpu.VMEM((CB,), jnp.int32),
            pltpu.VMEM((NIDX,), jnp.int32), pltpu.VMEM((NIDX,), jnp.int32),
            pltpu.VMEM((CB, DP), jnp.float32), pltpu.VMEM((CB, DP), jnp.float32),
            pltpu.VMEM((CB, DP), jnp.float32), pltpu.VMEM((CB, DP), jnp.float32),
            pltpu.VMEM((NIDX, DP), jnp.float32),
            pltpu.VMEM((NIDX, DP), jnp.float32),
            pltpu.VMEM((LANES,), jnp.float32),
            pltpu.SemaphoreType.DMA, pltpu.SemaphoreType.DMA,
        ],
    )


def kernel(center_ids, pos_ids, neg_ids, emb_in_weight, emb_out_weight):
    relayout = _build_tc_relayout()
    t_in = relayout(emb_in_weight.T, emb_in_weight.T)
    t_out = relayout(emb_out_weight.T, emb_out_weight.T)
    neg_flat = neg_ids.astype(jnp.int32).reshape(B * K)
    partials = _build_sc_kernel()(center_ids.astype(jnp.int32),
                                  pos_ids.astype(jnp.int32),
                                  neg_flat, t_in, t_out)
    return -jnp.sum(partials) / B
